# Initial kernel scaffold; baseline (speedup 1.0000x reference)
import jax
import jax.numpy as jnp
from jax.experimental import pallas as pl


def kernel(x, adj, lengs, size, s1_wl, s1_wr, s1_b, s2_wl, s2_wr, s2_b, s3_wl, s3_wr, s3_b, s4_wl, s4_wr, s4_b, tr1_w, tr1_b, tr2_w, tr2_b, dec0_w, dec0_b, dec1_w, dec1_b, d2_wl, d2_wr, d2_b, d3_wl, d3_wr, d3_b, d4_wl, d4_wr, d4_b, d5_wl, d5_wr, d5_b, bn1_g, bn1_b, bn2_g, bn2_b, bn3_g, bn3_b, bn4_g, bn4_b, bn5_g, bn5_b):
    raise NotImplementedError("write your pallas kernel here")


# speedup vs baseline: 2.3307x; 2.3307x over previous
import functools

import jax
import jax.numpy as jnp
from jax import lax
from jax.experimental import pallas as pl
from jax.experimental.pallas import tpu as pltpu
from jax.experimental.pallas import tpu_sc as plsc

_EPS = 1e-5
_NC = 2
_NS = 16
_CHUNK = 128


def _ceil_to(a, m):
    return (a + m - 1) // m * m


@functools.lru_cache(maxsize=None)
def _make_seg_sum(n_pad, d, n_chunks):
    half = n_pad // _NC
    acc_rows = half + 8 * _NS
    ept = n_chunks * _CHUNK
    rpt = half // _NS
    rpz = acc_rows // _NS
    mesh = plsc.VectorSubcoreMesh(core_axis_name="c", subcore_axis_name="s")

    out_type = jax.ShapeDtypeStruct((_NC, half, d), jnp.float32)
    scratch = [
        pltpu.VMEM((_CHUNK,), jnp.int32),
        pltpu.VMEM((_CHUNK,), jnp.int32),
        pltpu.VMEM((_CHUNK, d), jnp.float32),
        pltpu.VMEM((rpz, d), jnp.float32),
        pltpu.VMEM_SHARED((acc_rows, d), jnp.float32),
        pltpu.SemaphoreType.DMA,
    ]

    def body(tab_hbm, src_hbm, dst_hbm, z_hbm, out_hbm,
             src_v, dst_v, rows_v, stage_v, acc_sp, sem):
        s = lax.axis_index("s")
        c = lax.axis_index("c")
        e0 = s * ept

        pltpu.sync_copy(z_hbm.at[pl.ds(s * rpz, rpz)], stage_v)
        pltpu.sync_copy(stage_v, acc_sp.at[pl.ds(s * rpz, rpz)])
        plsc.subcore_barrier()

        def step(i, carry):
            base = pl.multiple_of(e0 + i * _CHUNK, _CHUNK)
            pltpu.sync_copy(src_hbm.at[pl.ds(base, _CHUNK)], src_v)
            pltpu.sync_copy(dst_hbm.at[c, pl.ds(base, _CHUNK)], dst_v)
            pltpu.async_copy(tab_hbm.at[src_v], rows_v, sem).wait()
            pltpu.sync_copy(rows_v, acc_sp.at[dst_v], add=True)
            return carry

        lax.fori_loop(0, n_chunks, step, 0)
        plsc.subcore_barrier()

        st = stage_v.at[pl.ds(0, rpt)]
        pltpu.sync_copy(acc_sp.at[pl.ds(s * rpt, rpt)], st)
        pltpu.sync_copy(st, out_hbm.at[c, pl.ds(s * rpt, rpt)])

    return pl.kernel(body, out_type=out_type, mesh=mesh,
                     scratch_types=scratch)


def _row_mask(shape, n_valid):
    return lax.broadcasted_iota(jnp.int32, shape, 0) < n_valid


def _bn_relu(y, g, b, n_valid, relu=True):
    mask = _row_mask(y.shape, n_valid)
    m = jnp.sum(y, axis=0, keepdims=True) * (1.0 / n_valid)
    yc = jnp.where(mask, y - m, 0.0)
    v = jnp.sum(yc * yc, axis=0, keepdims=True) * (1.0 / n_valid)
    o = yc * (g * lax.rsqrt(v + _EPS)) + b
    if relu:
        o = jnp.maximum(o, 0.0)
    return jnp.where(mask, o, 0.0)


def _sage_mm(n_valid, bn):
    def f(h_ref, agg_ref, deg_ref, wl_ref, wr_ref, g_ref, b_ref, o_ref):
        inv = 1.0 / jnp.maximum(deg_ref[...], 1.0)
        y = jnp.dot(agg_ref[...] * inv, wl_ref[...],
                    preferred_element_type=jnp.float32)
        y = y + jnp.dot(h_ref[...], wr_ref[...],
                        preferred_element_type=jnp.float32)
        if bn:
            o_ref[...] = _bn_relu(y, g_ref[...], b_ref[...], n_valid)
        else:
            mask = _row_mask(y.shape, n_valid)
            o_ref[...] = jnp.where(mask, y + b_ref[...], 0.0)
    return f


def _lin_bn(n_valid):
    def f(h_ref, w_ref, g_ref, b_ref, o_ref):
        y = jnp.dot(h_ref[...], w_ref[...], preferred_element_type=jnp.float32)
        o_ref[...] = _bn_relu(y, g_ref[...], b_ref[...], n_valid)
    return f


def _tr2_dec0(n_valid, rb):
    def f(h_ref, w2_ref, b2_ref, w0_ref, mu_ref, t_ref):
        i = pl.program_id(0)
        z = jnp.dot(h_ref[...], w2_ref[...],
                    preferred_element_type=jnp.float32) + b2_ref[...]
        mu_ref[...] = z
        mask = (lax.broadcasted_iota(jnp.int32, z.shape, 0) + i * rb) < n_valid
        zm = jnp.where(mask, z, 0.0)
        t_ref[...] = jnp.dot(zm, w0_ref[...],
                             preferred_element_type=jnp.float32)
    return f


def _dec_head(n_valid):
    def f(t_ref, g5_ref, b5_ref, w1_ref, g4_ref, b4_ref, o_ref):
        a = _bn_relu(t_ref[...], g5_ref[...], b5_ref[...], n_valid)
        y = jnp.dot(a, w1_ref[...], preferred_element_type=jnp.float32)
        o_ref[...] = _bn_relu(y, g4_ref[...], b4_ref[...], n_valid)
    return f


def kernel(x, adj, lengs, size, s1_wl, s1_wr, s1_b, s2_wl, s2_wr, s2_b,
           s3_wl, s3_wr, s3_b, s4_wl, s4_wr, s4_b, tr1_w, tr1_b, tr2_w,
           tr2_b, dec0_w, dec0_b, dec1_w, dec1_b, d2_wl, d2_wr, d2_b,
           d3_wl, d3_wr, d3_b, d4_wl, d4_wr, d4_b, d5_wl, d5_wr, d5_b,
           bn1_g, bn1_b, bn2_g, bn2_b, bn3_g, bn3_b, bn4_g, bn4_b,
           bn5_g, bn5_b):
    f32 = jnp.float32
    n, d = x.shape
    e = adj.shape[1]
    n_pad = _ceil_to(n + 1, 2 * _NS * 8)
    half = n_pad // _NC
    n_chunks = -(-e // (_NS * _CHUNK))
    e_pad = _NS * n_chunks * _CHUNK

    pad_idx = jnp.full((e_pad - e,), n, jnp.int32)
    src = jnp.concatenate([adj[0], pad_idx])
    dst = jnp.concatenate([adj[1], pad_idx])
    loc0 = jnp.where(dst < half, dst, half)
    l1 = dst - half
    loc1 = jnp.where(l1 >= 0, l1, half)
    dst2 = jnp.stack([loc0, loc1])
    zeros_acc = jnp.zeros((half + 8 * _NS, d), f32)
    ones_tab = jnp.ones((n_pad, d), f32)

    x_p = jnp.zeros((n_pad, d), f32).at[:n].set(x)

    seg_sum = _make_seg_sum(n_pad, d, n_chunks)

    def seg(h):
        return seg_sum(h, src, dst2, zeros_acc).reshape(n_pad, d)

    def r2(v):
        return v.reshape(1, -1)

    def sage(h, agg, wl, wr, b, g, beta, bn=True):
        gb = (r2(g), r2(beta)) if bn else (r2(b), r2(b))
        return pl.pallas_call(
            _sage_mm(n, bn),
            out_shape=jax.ShapeDtypeStruct((n_pad, d), f32),
        )(h, agg, deg, wl.T, wr.T, *gb)

    deg = seg(ones_tab)[:, 0:1]

    h = sage(x_p, seg(x_p), s1_wl, s1_wr, s1_b, bn1_g, bn1_b)
    h = sage(h, seg(h), s2_wl, s2_wr, s2_b, bn2_g, bn2_b)
    h = sage(h, seg(h), s3_wl, s3_wr, s3_b, bn3_g, bn3_b)
    h = sage(h, seg(h), s4_wl, s4_wr, s4_b, bn4_g, bn4_b)

    k1 = tr1_w.shape[0]
    h = pl.pallas_call(
        _lin_bn(n),
        out_shape=jax.ShapeDtypeStruct((n_pad, k1), f32),
    )(h, tr1_w.T, r2(bn5_g), r2(bn5_b))

    k2 = tr2_w.shape[0]
    nb = 8
    rb = n_pad // nb
    mu_full, t = pl.pallas_call(
        _tr2_dec0(n, rb),
        grid=(nb,),
        in_specs=[
            pl.BlockSpec((rb, k1), lambda i: (i, 0)),
            pl.BlockSpec((k1, k2), lambda i: (0, 0)),
            pl.BlockSpec((1, k2), lambda i: (0, 0)),
            pl.BlockSpec((k2, k1), lambda i: (0, 0)),
        ],
        out_specs=[
            pl.BlockSpec((rb, k2), lambda i: (i, 0)),
            pl.BlockSpec((rb, k1), lambda i: (i, 0)),
        ],
        out_shape=[jax.ShapeDtypeStruct((n_pad, k2), f32),
                   jax.ShapeDtypeStruct((n_pad, k1), f32)],
    )(h, tr2_w.T, r2(tr2_b), dec0_w.T)

    o = pl.pallas_call(
        _dec_head(n),
        out_shape=jax.ShapeDtypeStruct((n_pad, d), f32),
    )(t, r2(bn5_g), r2(bn5_b), dec1_w.T, r2(bn4_g), r2(bn4_b))

    o = sage(o, seg(o), d2_wl, d2_wr, d2_b, bn3_g, bn3_b)
    o = sage(o, seg(o), d3_wl, d3_wr, d3_b, bn2_g, bn2_b)
    o = sage(o, seg(o), d4_wl, d4_wr, d4_b, bn1_g, bn1_b)
    z2 = sage(o, seg(o), d5_wl, d5_wr, d5_b, None, None, bn=False)

    return z2[:n], mu_full[:n], mu_full[:n]
